# baseline (speedup 1.0000x reference)
import functools

import jax
import jax.numpy as jnp
from jax import lax
from jax.experimental import pallas as pl
from jax.experimental.pallas import tpu as pltpu
from jax.experimental.pallas import tpu_sc as plsc

NC, NS, L = 2, 16, 16
NW = NC * NS
BATCH, INPUTS, OUTPUTS = 1024, 16384, 32768
ROWS_PER_W = BATCH // NW
PAIRS = ROWS_PER_W // 2
CH = 8192
NCH = OUTPUTS // CH
NOB = 3


def _gather_body(x_hbm, idx_hbm, out_hbm, idx_v,
                 row0_v, row1_v, row2_v, row3_v,
                 ob0_v, ob1_v, ob2_v,
                 in_sem0, in_sem1, in_sem2, in_sem3,
                 ob_sem0, ob_sem1, ob_sem2):
    wid = lax.axis_index("s") * NC + lax.axis_index("c")
    base = wid * ROWS_PER_W
    pltpu.sync_copy(idx_hbm, idx_v)

    rows = ((row0_v, row1_v), (row2_v, row3_v))
    in_sems = ((in_sem0, in_sem1), (in_sem2, in_sem3))
    obs = (ob0_v, ob1_v, ob2_v)
    ob_sems = (ob_sem0, ob_sem1, ob_sem2)
    in_copies = [[None, None], [None, None]]
    ob_copies = [None] * NOB
    t = 0

    for k in range(2):
        in_copies[0][k] = pltpu.async_copy(
            x_hbm.at[base + k], rows[0][k], in_sems[0][k])
    for p in range(PAIRS):
        cur = p & 1
        if p + 1 < PAIRS:
            for k in range(2):
                in_copies[1 - cur][k] = pltpu.async_copy(
                    x_hbm.at[base + 2 * (p + 1) + k],
                    rows[1 - cur][k], in_sems[1 - cur][k])
        for k in range(2):
            in_copies[cur][k].wait()
        row_a, row_b = rows[cur]
        for c in range(NCH):
            ba, bb = t % NOB, (t + 1) % NOB
            t += 2
            for b in (ba, bb):
                if ob_copies[b] is not None:
                    ob_copies[b].wait()
            out_a, out_b = obs[ba], obs[bb]

            @plsc.parallel_loop(0, CH, step=L, unroll=4)
            def _chunk(j, c=c, row_a=row_a, row_b=row_b,
                       out_a=out_a, out_b=out_b):
                idx = idx_v[pl.ds(c * CH + j, L)]
                out_a[pl.ds(j, L)] = plsc.load_gather(row_a, [idx])
                out_b[pl.ds(j, L)] = plsc.load_gather(row_b, [idx])

            for k, b in ((0, ba), (1, bb)):
                ob_copies[b] = pltpu.async_copy(
                    obs[b],
                    out_hbm.at[base + 2 * p + k, pl.ds(c * CH, CH)],
                    ob_sems[b])
    for b in range(NOB):
        if ob_copies[b] is not None:
            ob_copies[b].wait()


_gather_call = functools.partial(
    pl.kernel,
    out_type=jax.ShapeDtypeStruct((BATCH, OUTPUTS), jnp.float32),
    mesh=plsc.VectorSubcoreMesh(
        core_axis_name="c", subcore_axis_name="s",
        num_cores=NC, num_subcores=NS,
    ),
    scratch_types=[
        pltpu.VMEM((OUTPUTS,), jnp.int32),
        pltpu.VMEM((INPUTS,), jnp.float32),
        pltpu.VMEM((INPUTS,), jnp.float32),
        pltpu.VMEM((INPUTS,), jnp.float32),
        pltpu.VMEM((INPUTS,), jnp.float32),
        pltpu.VMEM((CH,), jnp.float32),
        pltpu.VMEM((CH,), jnp.float32),
        pltpu.VMEM((CH,), jnp.float32),
        pltpu.SemaphoreType.DMA,
        pltpu.SemaphoreType.DMA,
        pltpu.SemaphoreType.DMA,
        pltpu.SemaphoreType.DMA,
        pltpu.SemaphoreType.DMA,
        pltpu.SemaphoreType.DMA,
        pltpu.SemaphoreType.DMA,
    ],
    compiler_params=pltpu.CompilerParams(needs_layout_passes=False),
)(_gather_body)


def kernel(x, indices):
    return _gather_call(x, indices)


# speedup vs baseline: 1.4116x; 1.4116x over previous
import functools

import jax
import jax.numpy as jnp
from jax import lax
from jax.experimental import pallas as pl
from jax.experimental.pallas import tpu as pltpu
from jax.experimental.pallas import tpu_sc as plsc

NC, NS, L = 2, 16, 16
NW = NC * NS
BATCH, INPUTS, OUTPUTS = 1024, 16384, 32768
ROWS_PER_W = BATCH // NW
HALF = OUTPUTS // 2
HWORDS = HALF // 2


def _gather_body(x_hbm, idx_hbm, out_hbm, idx_v, row0_v, row1_v,
                 outa_v, outb_v, in_sem0, in_sem1, out_sem0, out_sem1):
    wid = lax.axis_index("s") * NC + lax.axis_index("c")
    base = wid * ROWS_PER_W
    pltpu.sync_copy(idx_hbm, idx_v)

    rows = (row0_v, row1_v)
    outs = (outa_v, outb_v)
    in_sems = (in_sem0, in_sem1)
    out_sems = (out_sem0, out_sem1)
    in_copies = [None, None]
    out_copies = [None, None]

    in_copies[0] = pltpu.async_copy(x_hbm.at[base], rows[0], in_sems[0])
    for r in range(ROWS_PER_W):
        cur = r & 1
        if r + 1 < ROWS_PER_W:
            in_copies[1 - cur] = pltpu.async_copy(
                x_hbm.at[base + r + 1], rows[1 - cur], in_sems[1 - cur])
        in_copies[cur].wait()
        for h in range(2):
            if out_copies[h] is not None:
                out_copies[h].wait()
            row_ref = rows[cur]
            out_ref = outs[h]

            @plsc.parallel_loop(0, HWORDS, step=L, unroll=8)
            def _chunk(w, h=h, row_ref=row_ref, out_ref=out_ref):
                v = idx_v[pl.ds(h * HWORDS + w, L)]
                lo = v & 0xFFFF
                hi = v >> 16
                out_ref[pl.ds(2 * w, L)] = plsc.load_gather(row_ref, [lo])
                out_ref[pl.ds(2 * w + L, L)] = plsc.load_gather(row_ref, [hi])

            out_copies[h] = pltpu.async_copy(
                out_ref, out_hbm.at[base + r, pl.ds(h * HALF, HALF)],
                out_sems[h])
    for h in range(2):
        out_copies[h].wait()


_gather_call = functools.partial(
    pl.kernel,
    out_type=jax.ShapeDtypeStruct((BATCH, OUTPUTS), jnp.float32),
    mesh=plsc.VectorSubcoreMesh(
        core_axis_name="c", subcore_axis_name="s",
        num_cores=NC, num_subcores=NS,
    ),
    scratch_types=[
        pltpu.VMEM((OUTPUTS // 2,), jnp.int32),
        pltpu.VMEM((INPUTS,), jnp.float32),
        pltpu.VMEM((INPUTS,), jnp.float32),
        pltpu.VMEM((HALF,), jnp.float32),
        pltpu.VMEM((HALF,), jnp.float32),
        pltpu.SemaphoreType.DMA,
        pltpu.SemaphoreType.DMA,
        pltpu.SemaphoreType.DMA,
        pltpu.SemaphoreType.DMA,
    ],
    compiler_params=pltpu.CompilerParams(needs_layout_passes=False),
)(_gather_body)


def kernel(x, indices):
    u = indices.astype(jnp.uint32)
    blk = u.reshape(-1, 2, L)
    packed = blk[:, 0, :] | (blk[:, 1, :] << 16)
    idx_words = packed.reshape(-1).astype(jnp.int32)
    return _gather_call(x, idx_words)


# speedup vs baseline: 1.4359x; 1.0173x over previous
import functools

import jax
import jax.numpy as jnp
from jax import lax
from jax.experimental import pallas as pl
from jax.experimental.pallas import tpu as pltpu
from jax.experimental.pallas import tpu_sc as plsc

NC, NS, L = 2, 16, 16
NW = NC * NS
BATCH, INPUTS, OUTPUTS = 1024, 16384, 32768
ROWS_PER_W = BATCH // NW
HALF = OUTPUTS // 2
HWORDS = HALF // 2


def _gather_body(x_hbm, idx_hbm, out_hbm, idx_v, row0_v, row1_v,
                 outa_v, outb_v, outc_v, in_sem0, in_sem1, idx_sem,
                 out_sem0, out_sem1, out_sem2):
    wid = lax.axis_index("s") * NC + lax.axis_index("c")
    base = wid * ROWS_PER_W

    rows = (row0_v, row1_v)
    outs = (outa_v, outb_v, outc_v)
    in_sems = (in_sem0, in_sem1)
    out_sems = (out_sem0, out_sem1, out_sem2)
    in_copies = [None, None]
    out_copies = [None, None, None]

    idx_copy = pltpu.async_copy(idx_hbm, idx_v, idx_sem)
    in_copies[0] = pltpu.async_copy(x_hbm.at[base], rows[0], in_sems[0])
    idx_copy.wait()
    t = 0
    for r in range(ROWS_PER_W):
        cur = r & 1
        if r + 1 < ROWS_PER_W:
            in_copies[1 - cur] = pltpu.async_copy(
                x_hbm.at[base + r + 1], rows[1 - cur], in_sems[1 - cur])
        in_copies[cur].wait()
        for h in range(2):
            b = t % 3
            t += 1
            if out_copies[b] is not None:
                out_copies[b].wait()
            row_ref = rows[cur]
            out_ref = outs[b]

            @plsc.parallel_loop(0, HWORDS, step=L, unroll=8)
            def _chunk(w, h=h, row_ref=row_ref, out_ref=out_ref):
                v = idx_v[pl.ds(h * HWORDS + w, L)]
                lo = v & 0xFFFF
                hi = v >> 16
                out_ref[pl.ds(2 * w, L)] = plsc.load_gather(row_ref, [lo])
                out_ref[pl.ds(2 * w + L, L)] = plsc.load_gather(row_ref, [hi])

            out_copies[b] = pltpu.async_copy(
                out_ref, out_hbm.at[base + r, pl.ds(h * HALF, HALF)],
                out_sems[b])
    for b in range(3):
        if out_copies[b] is not None:
            out_copies[b].wait()


_gather_call = functools.partial(
    pl.kernel,
    out_type=jax.ShapeDtypeStruct((BATCH, OUTPUTS), jnp.float32),
    mesh=plsc.VectorSubcoreMesh(
        core_axis_name="c", subcore_axis_name="s",
        num_cores=NC, num_subcores=NS,
    ),
    scratch_types=[
        pltpu.VMEM((OUTPUTS // 2,), jnp.int32),
        pltpu.VMEM((INPUTS,), jnp.float32),
        pltpu.VMEM((INPUTS,), jnp.float32),
        pltpu.VMEM((HALF,), jnp.float32),
        pltpu.VMEM((HALF,), jnp.float32),
        pltpu.VMEM((HALF,), jnp.float32),
        pltpu.SemaphoreType.DMA,
        pltpu.SemaphoreType.DMA,
        pltpu.SemaphoreType.DMA,
        pltpu.SemaphoreType.DMA,
        pltpu.SemaphoreType.DMA,
        pltpu.SemaphoreType.DMA,
    ],
    compiler_params=pltpu.CompilerParams(needs_layout_passes=False),
)(_gather_body)


def kernel(x, indices):
    u = indices.astype(jnp.uint32)
    blk = u.reshape(-1, 2, L)
    packed = blk[:, 0, :] | (blk[:, 1, :] << 16)
    idx_words = packed.reshape(-1).astype(jnp.int32)
    return _gather_call(x, idx_words)
